# baseline (speedup 1.0000x reference)
import functools

import jax
import jax.numpy as jnp
from jax import lax
from jax.experimental import pallas as pl
from jax.experimental.pallas import tpu as pltpu
from jax.experimental.pallas import tpu_sc as plsc

N = 10000
E = 320000
B = 16
F = 128
M = 2500
P = 7500

NP_ = 10240
MP_ = 2560
NS = 16
NC = 2
WND = 128
NWIN = 160
G = 16
NGRP = NWIN // G
NBUF = 2
EPT = NWIN * WND
EP = EPT * NS
RPT = NP_ // NS
ZR = 32
PWND = 128
PWIN = 4
PPT = PWIN * PWND
PP = PPT * NS
PRT = MP_ // NS
BPC = B // NC
BN = 512

_f32 = jnp.float32
_mesh = plsc.VectorSubcoreMesh(core_axis_name="c", subcore_axis_name="s")


def _zero_fill_2d(ref, rows):
    def body(i, _):
        for f in range(F // 16):
            ref[i, pl.ds(f * 16, 16)] = jnp.zeros((16,), _f32)
        return 0
    lax.fori_loop(0, rows, body, 0)


@functools.partial(
    pl.kernel,
    out_type=jax.ShapeDtypeStruct((NC, NP_), _f32),
    mesh=_mesh,
    scratch_types=[
        pltpu.VMEM_SHARED((NP_,), _f32),
        pltpu.VMEM((NWIN, WND), jnp.int32),
        pltpu.VMEM((WND,), _f32),
        pltpu.VMEM((RPT,), _f32),
    ],
)
def _sc_deg(row3, deg2, acc1, row_v, ones_v, zeros_v):
    cid = lax.axis_index("c")
    tid = lax.axis_index("s")
    pltpu.sync_copy(row3.at[tid], row_v)

    def fill_ones(i, _):
        ones_v[pl.ds(i * 16, 16)] = jnp.ones((16,), _f32)
        return 0
    lax.fori_loop(0, WND // 16, fill_ones, 0)

    def fill_zeros(i, _):
        zeros_v[pl.ds(i * 16, 16)] = jnp.zeros((16,), _f32)
        return 0
    lax.fori_loop(0, RPT // 16, fill_zeros, 0)

    pltpu.sync_copy(zeros_v, acc1.at[pl.ds(tid * RPT, RPT)])
    plsc.subcore_barrier()

    half = NWIN // NC

    def win(w, _):
        pltpu.sync_copy(ones_v, acc1.at[row_v.at[cid * half + w]], add=True)
        return 0
    lax.fori_loop(0, half, win, 0)

    plsc.subcore_barrier()
    pltpu.sync_copy(acc1.at[pl.ds(tid * RPT, RPT)],
                    deg2.at[cid].at[pl.ds(tid * RPT, RPT)])


@functools.partial(
    pl.kernel,
    out_type=jax.ShapeDtypeStruct((B, NP_, F), _f32),
    mesh=_mesh,
    scratch_types=[
        pltpu.VMEM_SHARED((NP_, F), _f32),
        pltpu.VMEM((G, 2, WND), jnp.int32),
        pltpu.VMEM((G, 2, WND), jnp.int32),
        [pltpu.VMEM((WND, F), _f32)] * NBUF,
        [pltpu.SemaphoreType.DMA] * NBUF,
        [pltpu.SemaphoreType.DMA] * NBUF,
        pltpu.SemaphoreType.DMA,
        pltpu.SemaphoreType.DMA,
    ],
)
def _sc_prop(u, idx4, zeros2, g, acc, idxa, idxb, bufs, gsems, ssems,
             isema, isemb):
    cid = lax.axis_index("c")
    tid = lax.axis_index("s")
    row0 = tid * RPT
    idx_t = idx4.at[tid]

    def batch_body(b8, _):
        b = cid * BPC + b8
        pltpu.sync_copy(zeros2.at[pl.ds(row0, RPT)], acc.at[pl.ds(row0, RPT)])
        plsc.subcore_barrier()
        u_b = u.at[b]

        def gstart(ic, w, j):
            pltpu.make_async_copy(u_b.at[ic.at[w, 0]], bufs[j],
                                  gsems[j]).start()

        def gwait(ic, w, j):
            pltpu.make_async_copy(u_b.at[ic.at[w, 0]], bufs[j],
                                  gsems[j]).wait()

        def swait(ic, w, j):
            pltpu.make_async_copy(bufs[j], acc.at[ic.at[w, 1]],
                                  ssems[j]).wait()

        def process(ic):
            for j in range(NBUF):
                gstart(ic, j, j)

            def round_body(r, _):
                w0 = r * NBUF
                for j in range(NBUF):
                    gwait(ic, w0 + j, j)
                    pltpu.async_copy(bufs[j], acc.at[ic.at[w0 + j, 1]],
                                     ssems[j], add=True)

                @pl.when(r < G // NBUF - 1)
                def _():
                    for j in range(NBUF):
                        swait(ic, w0 + j, j)
                        gstart(ic, w0 + NBUF + j, j)
                return 0
            lax.fori_loop(0, G // NBUF, round_body, 0)
            for j in range(NBUF):
                swait(ic, G - NBUF + j, j)

        def istart(ic, isem, gi):
            pltpu.make_async_copy(idx_t.at[gi], ic, isem).start()

        def iwait(ic, isem, gi):
            pltpu.make_async_copy(idx_t.at[gi], ic, isem).wait()

        pltpu.sync_copy(idx_t.at[0], idxa)
        istart(idxb, isemb, 1)

        def gpair_body(q, _):
            g0 = 2 * q
            process(idxa)

            @pl.when(q < NGRP // 2 - 1)
            def _():
                istart(idxa, isema, g0 + 2)

            iwait(idxb, isemb, g0 + 1)
            process(idxb)

            @pl.when(q < NGRP // 2 - 1)
            def _():
                istart(idxb, isemb, g0 + 3)
                iwait(idxa, isema, g0 + 2)
            return 0
        lax.fori_loop(0, NGRP // 2, gpair_body, 0)

        plsc.subcore_barrier()
        pltpu.sync_copy(acc.at[pl.ds(row0, RPT)],
                        g.at[b].at[pl.ds(row0, RPT)])
        return 0
    lax.fori_loop(0, BPC, batch_body, 0)


@functools.partial(
    pl.kernel,
    out_type=jax.ShapeDtypeStruct((B, MP_, F), _f32),
    mesh=_mesh,
    scratch_types=[
        pltpu.VMEM_SHARED((MP_, F), _f32),
        pltpu.VMEM((PWIN, 2, PWND), jnp.int32),
        pltpu.VMEM((PWND, F), _f32),
        pltpu.VMEM((PWND, F), _f32),
        pltpu.VMEM((ZR, F), _f32),
    ],
)
def _sc_pool(y, pidx4, pval4, out, accp, pidx_v, valw, buf, zeros_v):
    cid = lax.axis_index("c")
    tid = lax.axis_index("s")
    pltpu.sync_copy(pidx4.at[tid], pidx_v)
    _zero_fill_2d(zeros_v, ZR)
    r0 = tid * PRT
    pval_t = pval4.at[tid]

    def batch_body(b8, _):
        b = cid * BPC + b8
        for z in range(PRT // ZR):
            pltpu.sync_copy(zeros_v, accp.at[pl.ds(r0 + z * ZR, ZR)])
        plsc.subcore_barrier()
        y_b = y.at[b]
        for w in range(PWIN):
            pltpu.sync_copy(y_b.at[pidx_v.at[w, 0]], buf)
            pltpu.sync_copy(pval_t.at[w], valw)

            def scale(i, _):
                for f in range(F // 16):
                    sl = pl.ds(f * 16, 16)
                    buf[i, sl] = buf[i, sl] * valw[i, sl]
                return 0
            lax.fori_loop(0, PWND, scale, 0)
            pltpu.sync_copy(buf, accp.at[pidx_v.at[w, 1]], add=True)
        plsc.subcore_barrier()
        pltpu.sync_copy(accp.at[pl.ds(r0, PRT)],
                        out.at[b].at[pl.ds(r0, PRT)])
        return 0
    lax.fori_loop(0, BPC, batch_body, 0)


def _mul2_body(a_ref, c_ref, o_ref):
    o_ref[...] = a_ref[...] * c_ref[...][None]


def _mulsub_body(g_ref, c_ref, v_ref, o_ref):
    o_ref[...] = c_ref[...][None] * g_ref[...] - v_ref[...]


_blk3 = lambda: pl.BlockSpec((1, BN, F), lambda bi, ni: (bi, ni, 0))
_blk2 = lambda: pl.BlockSpec((BN, F), lambda bi, ni: (ni, 0))
_out3 = jax.ShapeDtypeStruct((B, NP_, F), _f32)

_tc_mul2 = pl.pallas_call(
    _mul2_body, grid=(B, NP_ // BN),
    in_specs=[_blk3(), _blk2()], out_specs=_blk3(), out_shape=_out3)

_tc_mulsub = pl.pallas_call(
    _mulsub_body, grid=(B, NP_ // BN),
    in_specs=[_blk3(), _blk2(), _blk3()], out_specs=_blk3(), out_shape=_out3)


def _final_body(x_ref, g1_ref, g2_ref, g3_ref, g4_ref, g5_ref, dinv_ref,
                w_ref, b_ref, y_ref):
    w = w_ref[...]
    v0 = w[0] - w[2] + w[4]
    v1 = w[1] - w[3] + w[5]
    v2 = 2.0 * (w[2] - w[4])
    v3 = 2.0 * (w[3] - w[5])
    v4 = 2.0 * w[4]
    v5 = 2.0 * w[5]
    dot = lambda a, bm: jax.lax.dot(a, bm, preferred_element_type=_f32)
    accg = dot(g1_ref[0], v1)
    accg += dot(g2_ref[0], v2)
    accg += dot(g3_ref[0], v3)
    accg += dot(g4_ref[0], v4)
    accg += dot(g5_ref[0], v5)
    o = dot(x_ref[0], v0) + dinv_ref[...] * accg + b_ref[...]
    y_ref[0] = jnp.where(o > 0.0, o, jnp.exp(o) - 1.0)


_tc_final = pl.pallas_call(
    _final_body, grid=(B, NP_ // BN),
    in_specs=[_blk3(), _blk3(), _blk3(), _blk3(), _blk3(), _blk3(), _blk2(),
              pl.BlockSpec((6, F, F), lambda bi, ni: (0, 0, 0)),
              pl.BlockSpec((1, F), lambda bi, ni: (0, 0))],
    out_specs=_blk3(), out_shape=_out3)


def kernel(x, edge_index, down_row, down_col, down_val, W, b):
    row = edge_index[0]
    col = edge_index[1]

    pade = EP - E
    ar = jnp.arange(pade, dtype=jnp.int32)
    colp = jnp.concatenate([col, ar % N])
    rowp = jnp.concatenate([row, N + ar % (NP_ - N)])
    idx4 = jnp.stack(
        [colp.reshape(NS, NGRP, G, WND), rowp.reshape(NS, NGRP, G, WND)],
        axis=3)
    row3 = rowp.reshape(NS, NWIN, WND)
    x_pad = jnp.pad(x, ((0, 0), (0, NP_ - N), (0, 0)))

    deg2 = _sc_deg(row3)
    deg = deg2[0] + deg2[1]
    dinv = jnp.where(deg > 0.0, lax.rsqrt(deg), 0.0)
    dinv_b = jnp.broadcast_to(dinv[:, None], (NP_, F))
    c1_b = -(dinv_b * dinv_b)
    c2_b = 2.0 * c1_b
    zeros2 = jnp.zeros((NP_, F), _f32)

    v0 = _tc_mul2(x_pad, -dinv_b)
    g1 = _sc_prop(v0, idx4, zeros2)
    v1 = _tc_mul2(g1, c1_b)
    g2 = _sc_prop(v1, idx4, zeros2)
    v2 = _tc_mulsub(g2, c2_b, v0)
    g3 = _sc_prop(v2, idx4, zeros2)
    v3 = _tc_mulsub(g3, c2_b, v1)
    g4 = _sc_prop(v3, idx4, zeros2)
    v4 = _tc_mulsub(g4, c2_b, v2)
    g5 = _sc_prop(v4, idx4, zeros2)

    y = _tc_final(x_pad, g1, g2, g3, g4, g5, dinv_b, W, b.reshape(1, F))

    padp = PP - P
    arp = jnp.arange(padp, dtype=jnp.int32)
    pcol = jnp.concatenate([down_col, arp % N])
    prow = jnp.concatenate([down_row, M + arp % (MP_ - M)])
    pidx4 = jnp.stack(
        [pcol.reshape(NS, PWIN, PWND), prow.reshape(NS, PWIN, PWND)], axis=2)
    pval = jnp.concatenate([down_val, jnp.zeros((padp,), _f32)])
    pval4 = jnp.broadcast_to(pval[:, None], (PP, F)).reshape(NS, PWIN, PWND, F)

    pooled = _sc_pool(y, pidx4, pval4)
    return pooled[:, :M, :]


# speedup vs baseline: 1.2568x; 1.2568x over previous
import functools

import jax
import jax.numpy as jnp
from jax import lax
from jax.experimental import pallas as pl
from jax.experimental.pallas import tpu as pltpu
from jax.experimental.pallas import tpu_sc as plsc

N = 10000
E = 320000
B = 16
F = 128
M = 2500
P = 7500

NP_ = 10240
MP_ = 2560
NS = 16
NC = 2
WND = 128
NWIN = 160
G = 16
NGRP = NWIN // G
NBUF = 2
EPT = NWIN * WND
EP = EPT * NS
RPT = NP_ // NS
ZR = 32
PWND = 128
PWIN = 4
PPT = PWIN * PWND
PP = PPT * NS
PRT = MP_ // NS
BPC = B // NC
BN = 512

_f32 = jnp.float32
_mesh = plsc.VectorSubcoreMesh(core_axis_name="c", subcore_axis_name="s")


def _zero_fill_2d(ref, rows):
    def body(i, _):
        for f in range(F // 16):
            ref[i, pl.ds(f * 16, 16)] = jnp.zeros((16,), _f32)
        return 0
    lax.fori_loop(0, rows, body, 0)


@functools.partial(
    pl.kernel,
    out_type=jax.ShapeDtypeStruct((NC, NP_), _f32),
    mesh=_mesh,
    scratch_types=[
        pltpu.VMEM_SHARED((NP_,), _f32),
        pltpu.VMEM((NWIN, WND), jnp.int32),
        pltpu.VMEM((WND,), _f32),
        pltpu.VMEM((RPT,), _f32),
    ],
)
def _sc_deg(row3, deg2, acc1, row_v, ones_v, zeros_v):
    cid = lax.axis_index("c")
    tid = lax.axis_index("s")
    pltpu.sync_copy(row3.at[tid], row_v)

    def fill_ones(i, _):
        ones_v[pl.ds(i * 16, 16)] = jnp.ones((16,), _f32)
        return 0
    lax.fori_loop(0, WND // 16, fill_ones, 0)

    def fill_zeros(i, _):
        zeros_v[pl.ds(i * 16, 16)] = jnp.zeros((16,), _f32)
        return 0
    lax.fori_loop(0, RPT // 16, fill_zeros, 0)

    pltpu.sync_copy(zeros_v, acc1.at[pl.ds(tid * RPT, RPT)])
    plsc.subcore_barrier()

    half = NWIN // NC

    def win(w, _):
        pltpu.sync_copy(ones_v, acc1.at[row_v.at[cid * half + w]], add=True)
        return 0
    lax.fori_loop(0, half, win, 0)

    plsc.subcore_barrier()
    pltpu.sync_copy(acc1.at[pl.ds(tid * RPT, RPT)],
                    deg2.at[cid].at[pl.ds(tid * RPT, RPT)])


@functools.partial(
    pl.kernel,
    out_type=jax.ShapeDtypeStruct((B, NP_, F), _f32),
    mesh=_mesh,
    scratch_types=[
        pltpu.VMEM_SHARED((NP_, F), _f32),
        pltpu.VMEM((G, 2, WND), jnp.int32),
        pltpu.VMEM((G, 2, WND), jnp.int32),
        [pltpu.VMEM((WND, F), _f32)] * NBUF,
        [pltpu.SemaphoreType.DMA] * NBUF,
        pltpu.SemaphoreType.DMA,
        pltpu.SemaphoreType.DMA,
    ],
)
def _sc_prop(u, idx4, zeros2, g, acc, idxa, idxb, bufs, gsems,
             isema, isemb):
    cid = lax.axis_index("c")
    tid = lax.axis_index("s")
    row0 = tid * RPT
    idx_t = idx4.at[tid]

    def batch_body(b8, _):
        b = cid * BPC + b8
        pltpu.sync_copy(zeros2.at[pl.ds(row0, RPT)], acc.at[pl.ds(row0, RPT)])
        plsc.subcore_barrier()
        u_b = u.at[b]

        def gstart(ic, w, j):
            pltpu.make_async_copy(u_b.at[ic.at[w, 0]], bufs[j],
                                  gsems[j]).start()

        def gwait(ic, w, j):
            pltpu.make_async_copy(u_b.at[ic.at[w, 0]], bufs[j],
                                  gsems[j]).wait()

        def process(ic):
            gstart(ic, 0, 0)

            def pair(p, _):
                w0 = 2 * p
                gstart(ic, w0 + 1, 1)
                gwait(ic, w0, 0)
                pltpu.sync_copy(bufs[0], acc.at[ic.at[w0, 1]], add=True)

                @pl.when(p < (G // 2 - 1))
                def _():
                    gstart(ic, w0 + 2, 0)

                gwait(ic, w0 + 1, 1)
                pltpu.sync_copy(bufs[1], acc.at[ic.at[w0 + 1, 1]], add=True)
                return 0
            lax.fori_loop(0, G // 2, pair, 0)

        def istart(ic, isem, gi):
            pltpu.make_async_copy(idx_t.at[gi], ic, isem).start()

        def iwait(ic, isem, gi):
            pltpu.make_async_copy(idx_t.at[gi], ic, isem).wait()

        pltpu.sync_copy(idx_t.at[0], idxa)
        istart(idxb, isemb, 1)

        def gpair_body(q, _):
            g0 = 2 * q
            process(idxa)

            @pl.when(q < NGRP // 2 - 1)
            def _():
                istart(idxa, isema, g0 + 2)

            iwait(idxb, isemb, g0 + 1)
            process(idxb)

            @pl.when(q < NGRP // 2 - 1)
            def _():
                istart(idxb, isemb, g0 + 3)
                iwait(idxa, isema, g0 + 2)
            return 0
        lax.fori_loop(0, NGRP // 2, gpair_body, 0)

        plsc.subcore_barrier()
        pltpu.sync_copy(acc.at[pl.ds(row0, RPT)],
                        g.at[b].at[pl.ds(row0, RPT)])
        return 0
    lax.fori_loop(0, BPC, batch_body, 0)


@functools.partial(
    pl.kernel,
    out_type=jax.ShapeDtypeStruct((B, MP_, F), _f32),
    mesh=_mesh,
    scratch_types=[
        pltpu.VMEM_SHARED((MP_, F), _f32),
        pltpu.VMEM((PWIN, 2, PWND), jnp.int32),
        pltpu.VMEM((PWND, F), _f32),
        pltpu.VMEM((PWND, F), _f32),
        pltpu.VMEM((ZR, F), _f32),
    ],
)
def _sc_pool(y, pidx4, pval4, out, accp, pidx_v, valw, buf, zeros_v):
    cid = lax.axis_index("c")
    tid = lax.axis_index("s")
    pltpu.sync_copy(pidx4.at[tid], pidx_v)
    _zero_fill_2d(zeros_v, ZR)
    r0 = tid * PRT
    pval_t = pval4.at[tid]

    def batch_body(b8, _):
        b = cid * BPC + b8
        for z in range(PRT // ZR):
            pltpu.sync_copy(zeros_v, accp.at[pl.ds(r0 + z * ZR, ZR)])
        plsc.subcore_barrier()
        y_b = y.at[b]
        for w in range(PWIN):
            pltpu.sync_copy(y_b.at[pidx_v.at[w, 0]], buf)
            pltpu.sync_copy(pval_t.at[w], valw)

            def scale(i, _):
                for f in range(F // 16):
                    sl = pl.ds(f * 16, 16)
                    buf[i, sl] = buf[i, sl] * valw[i, sl]
                return 0
            lax.fori_loop(0, PWND, scale, 0)
            pltpu.sync_copy(buf, accp.at[pidx_v.at[w, 1]], add=True)
        plsc.subcore_barrier()
        pltpu.sync_copy(accp.at[pl.ds(r0, PRT)],
                        out.at[b].at[pl.ds(r0, PRT)])
        return 0
    lax.fori_loop(0, BPC, batch_body, 0)


def _mul2_body(a_ref, c_ref, o_ref):
    o_ref[...] = a_ref[...] * c_ref[...][None]


def _mulsub_body(g_ref, c_ref, v_ref, o_ref):
    o_ref[...] = c_ref[...][None] * g_ref[...] - v_ref[...]


_blk3 = lambda: pl.BlockSpec((1, BN, F), lambda bi, ni: (bi, ni, 0))
_blk2 = lambda: pl.BlockSpec((BN, F), lambda bi, ni: (ni, 0))
_out3 = jax.ShapeDtypeStruct((B, NP_, F), _f32)

_tc_mul2 = pl.pallas_call(
    _mul2_body, grid=(B, NP_ // BN),
    in_specs=[_blk3(), _blk2()], out_specs=_blk3(), out_shape=_out3)

_tc_mulsub = pl.pallas_call(
    _mulsub_body, grid=(B, NP_ // BN),
    in_specs=[_blk3(), _blk2(), _blk3()], out_specs=_blk3(), out_shape=_out3)


def _final_body(x_ref, g1_ref, g2_ref, g3_ref, g4_ref, g5_ref, dinv_ref,
                w_ref, b_ref, y_ref):
    w = w_ref[...]
    v0 = w[0] - w[2] + w[4]
    v1 = w[1] - w[3] + w[5]
    v2 = 2.0 * (w[2] - w[4])
    v3 = 2.0 * (w[3] - w[5])
    v4 = 2.0 * w[4]
    v5 = 2.0 * w[5]
    dot = lambda a, bm: jax.lax.dot(a, bm, preferred_element_type=_f32)
    accg = dot(g1_ref[0], v1)
    accg += dot(g2_ref[0], v2)
    accg += dot(g3_ref[0], v3)
    accg += dot(g4_ref[0], v4)
    accg += dot(g5_ref[0], v5)
    o = dot(x_ref[0], v0) + dinv_ref[...] * accg + b_ref[...]
    y_ref[0] = jnp.where(o > 0.0, o, jnp.exp(o) - 1.0)


_tc_final = pl.pallas_call(
    _final_body, grid=(B, NP_ // BN),
    in_specs=[_blk3(), _blk3(), _blk3(), _blk3(), _blk3(), _blk3(), _blk2(),
              pl.BlockSpec((6, F, F), lambda bi, ni: (0, 0, 0)),
              pl.BlockSpec((1, F), lambda bi, ni: (0, 0))],
    out_specs=_blk3(), out_shape=_out3)


def kernel(x, edge_index, down_row, down_col, down_val, W, b):
    row = edge_index[0]
    col = edge_index[1]

    pade = EP - E
    ar = jnp.arange(pade, dtype=jnp.int32)
    colp = jnp.concatenate([col, ar % N])
    rowp = jnp.concatenate([row, N + ar % (NP_ - N)])
    idx4 = jnp.stack(
        [colp.reshape(NS, NGRP, G, WND), rowp.reshape(NS, NGRP, G, WND)],
        axis=3)
    row3 = rowp.reshape(NS, NWIN, WND)
    x_pad = jnp.pad(x, ((0, 0), (0, NP_ - N), (0, 0)))

    deg2 = _sc_deg(row3)
    deg = deg2[0] + deg2[1]
    dinv = jnp.where(deg > 0.0, lax.rsqrt(deg), 0.0)
    dinv_b = jnp.broadcast_to(dinv[:, None], (NP_, F))
    c1_b = -(dinv_b * dinv_b)
    c2_b = 2.0 * c1_b
    zeros2 = jnp.zeros((NP_, F), _f32)

    v0 = _tc_mul2(x_pad, -dinv_b)
    g1 = _sc_prop(v0, idx4, zeros2)
    v1 = _tc_mul2(g1, c1_b)
    g2 = _sc_prop(v1, idx4, zeros2)
    v2 = _tc_mulsub(g2, c2_b, v0)
    g3 = _sc_prop(v2, idx4, zeros2)
    v3 = _tc_mulsub(g3, c2_b, v1)
    g4 = _sc_prop(v3, idx4, zeros2)
    v4 = _tc_mulsub(g4, c2_b, v2)
    g5 = _sc_prop(v4, idx4, zeros2)

    y = _tc_final(x_pad, g1, g2, g3, g4, g5, dinv_b, W, b.reshape(1, F))

    padp = PP - P
    arp = jnp.arange(padp, dtype=jnp.int32)
    pcol = jnp.concatenate([down_col, arp % N])
    prow = jnp.concatenate([down_row, M + arp % (MP_ - M)])
    pidx4 = jnp.stack(
        [pcol.reshape(NS, PWIN, PWND), prow.reshape(NS, PWIN, PWND)], axis=2)
    pval = jnp.concatenate([down_val, jnp.zeros((padp,), _f32)])
    pval4 = jnp.broadcast_to(pval[:, None], (PP, F)).reshape(NS, PWIN, PWND, F)

    pooled = _sc_pool(y, pidx4, pval4)
    return pooled[:, :M, :]


# speedup vs baseline: 1.2669x; 1.0080x over previous
import functools

import jax
import jax.numpy as jnp
from jax import lax
from jax.experimental import pallas as pl
from jax.experimental.pallas import tpu as pltpu
from jax.experimental.pallas import tpu_sc as plsc

N = 10000
E = 320000
B = 16
F = 128
M = 2500
P = 7500

NP_ = 10240
MP_ = 2560
NS = 16
NC = 2
WND = 128
NWIN = 160
G = 16
NGRP = NWIN // G
NBUF = 2
EPT = NWIN * WND
EP = EPT * NS
RPT = NP_ // NS
ZR = 32
PWND = 128
PWIN = 4
PPT = PWIN * PWND
PP = PPT * NS
PRT = MP_ // NS
BPC = B // NC
BN = 512

_f32 = jnp.float32
_mesh = plsc.VectorSubcoreMesh(core_axis_name="c", subcore_axis_name="s")


def _zero_fill_2d(ref, rows):
    def body(i, _):
        for f in range(F // 16):
            ref[i, pl.ds(f * 16, 16)] = jnp.zeros((16,), _f32)
        return 0
    lax.fori_loop(0, rows, body, 0)


@functools.partial(
    pl.kernel,
    out_type=jax.ShapeDtypeStruct((NC, NP_), _f32),
    mesh=_mesh,
    scratch_types=[
        pltpu.VMEM_SHARED((NP_,), _f32),
        pltpu.VMEM((NWIN, WND), jnp.int32),
        pltpu.VMEM((WND,), _f32),
        pltpu.VMEM((RPT,), _f32),
    ],
)
def _sc_deg(row3, deg2, acc1, row_v, ones_v, zeros_v):
    cid = lax.axis_index("c")
    tid = lax.axis_index("s")
    pltpu.sync_copy(row3.at[tid], row_v)

    def fill_ones(i, _):
        ones_v[pl.ds(i * 16, 16)] = jnp.ones((16,), _f32)
        return 0
    lax.fori_loop(0, WND // 16, fill_ones, 0)

    def fill_zeros(i, _):
        zeros_v[pl.ds(i * 16, 16)] = jnp.zeros((16,), _f32)
        return 0
    lax.fori_loop(0, RPT // 16, fill_zeros, 0)

    pltpu.sync_copy(zeros_v, acc1.at[pl.ds(tid * RPT, RPT)])
    plsc.subcore_barrier()

    half = NWIN // NC

    def win(w, _):
        pltpu.sync_copy(ones_v, acc1.at[row_v.at[cid * half + w]], add=True)
        return 0
    lax.fori_loop(0, half, win, 0)

    plsc.subcore_barrier()
    pltpu.sync_copy(acc1.at[pl.ds(tid * RPT, RPT)],
                    deg2.at[cid].at[pl.ds(tid * RPT, RPT)])


@functools.partial(
    pl.kernel,
    out_type=jax.ShapeDtypeStruct((B, NP_, F), _f32),
    mesh=_mesh,
    scratch_types=[
        pltpu.VMEM_SHARED((NP_, F), _f32),
        pltpu.VMEM((G, 2, WND), jnp.int32),
        pltpu.VMEM((G, 2, WND), jnp.int32),
        [pltpu.VMEM((WND, F), _f32)] * NBUF,
        [pltpu.SemaphoreType.DMA] * NBUF,
        pltpu.SemaphoreType.DMA,
        pltpu.SemaphoreType.DMA,
    ],
)
def _sc_prop(u, idx4, zeros2, g, acc, idxa, idxb, bufs, gsems,
             isema, isemb):
    cid = lax.axis_index("c")
    tid = lax.axis_index("s")
    row0 = tid * RPT
    idx_t = idx4.at[tid]

    def batch_body(b8, _):
        b = cid * BPC + b8
        pltpu.sync_copy(zeros2.at[pl.ds(row0, RPT)], acc.at[pl.ds(row0, RPT)])
        plsc.subcore_barrier()
        u_b = u.at[b]

        def gstart(ic, w, j):
            pltpu.make_async_copy(u_b.at[ic.at[w, 0]], bufs[j],
                                  gsems[j]).start()

        def gwait(ic, w, j):
            pltpu.make_async_copy(u_b.at[ic.at[w, 0]], bufs[j],
                                  gsems[j]).wait()

        def process(ic):
            gstart(ic, 0, 0)

            def pair(p, _):
                w0 = 2 * p
                gstart(ic, w0 + 1, 1)
                gwait(ic, w0, 0)
                pltpu.sync_copy(bufs[0], acc.at[ic.at[w0, 1]], add=True)

                @pl.when(p < (G // 2 - 1))
                def _():
                    gstart(ic, w0 + 2, 0)

                gwait(ic, w0 + 1, 1)
                pltpu.sync_copy(bufs[1], acc.at[ic.at[w0 + 1, 1]], add=True)
                return 0
            lax.fori_loop(0, G // 2, pair, 0)

        def istart(ic, isem, gi):
            pltpu.make_async_copy(idx_t.at[gi], ic, isem).start()

        def iwait(ic, isem, gi):
            pltpu.make_async_copy(idx_t.at[gi], ic, isem).wait()

        pltpu.sync_copy(idx_t.at[0], idxa)
        istart(idxb, isemb, 1)

        def gpair_body(q, _):
            g0 = 2 * q
            process(idxa)

            @pl.when(q < NGRP // 2 - 1)
            def _():
                istart(idxa, isema, g0 + 2)

            iwait(idxb, isemb, g0 + 1)
            process(idxb)

            @pl.when(q < NGRP // 2 - 1)
            def _():
                istart(idxb, isemb, g0 + 3)
                iwait(idxa, isema, g0 + 2)
            return 0
        lax.fori_loop(0, NGRP // 2, gpair_body, 0)

        plsc.subcore_barrier()
        pltpu.sync_copy(acc.at[pl.ds(row0, RPT)],
                        g.at[b].at[pl.ds(row0, RPT)])
        return 0
    lax.fori_loop(0, BPC, batch_body, 0)


@functools.partial(
    pl.kernel,
    out_type=jax.ShapeDtypeStruct((B, MP_, F), _f32),
    mesh=_mesh,
    scratch_types=[
        pltpu.VMEM_SHARED((MP_, F), _f32),
        pltpu.VMEM((PWIN, 2, PWND), jnp.int32),
        pltpu.VMEM((PWND, F), _f32),
        pltpu.VMEM((PWND, F), _f32),
        pltpu.VMEM((ZR, F), _f32),
    ],
)
def _sc_pool(y, pidx4, pval4, out, accp, pidx_v, valw, buf, zeros_v):
    cid = lax.axis_index("c")
    tid = lax.axis_index("s")
    pltpu.sync_copy(pidx4.at[tid], pidx_v)
    _zero_fill_2d(zeros_v, ZR)
    r0 = tid * PRT
    pval_t = pval4.at[tid]

    def batch_body(b8, _):
        b = cid * BPC + b8
        for z in range(PRT // ZR):
            pltpu.sync_copy(zeros_v, accp.at[pl.ds(r0 + z * ZR, ZR)])
        plsc.subcore_barrier()
        y_b = y.at[b]
        for w in range(PWIN):
            pltpu.sync_copy(y_b.at[pidx_v.at[w, 0]], buf)
            pltpu.sync_copy(pval_t.at[w], valw)

            def scale(i, _):
                for f in range(F // 16):
                    sl = pl.ds(f * 16, 16)
                    buf[i, sl] = buf[i, sl] * valw[i, sl]
                return 0
            lax.fori_loop(0, PWND, scale, 0)
            pltpu.sync_copy(buf, accp.at[pidx_v.at[w, 1]], add=True)
        plsc.subcore_barrier()
        pltpu.sync_copy(accp.at[pl.ds(r0, PRT)],
                        out.at[b].at[pl.ds(r0, PRT)])
        return 0
    lax.fori_loop(0, BPC, batch_body, 0)


def _mul2_body(a_ref, c_ref, o_ref):
    o_ref[...] = a_ref[...] * c_ref[...][None]


def _mulsub_body(g_ref, c_ref, v_ref, o_ref):
    o_ref[...] = c_ref[...][None] * g_ref[...] - v_ref[...]


_blk3 = lambda: pl.BlockSpec((1, BN, F), lambda ni, bi: (bi, ni, 0))
_blk2 = lambda: pl.BlockSpec((BN, F), lambda ni, bi: (ni, 0))
_out3 = jax.ShapeDtypeStruct((B, NP_, F), _f32)

_tc_mul2 = pl.pallas_call(
    _mul2_body, grid=(NP_ // BN, B),
    in_specs=[_blk3(), _blk2()], out_specs=_blk3(), out_shape=_out3)

_tc_mulsub = pl.pallas_call(
    _mulsub_body, grid=(NP_ // BN, B),
    in_specs=[_blk3(), _blk2(), _blk3()], out_specs=_blk3(), out_shape=_out3)


def _final_body(x_ref, g1_ref, g2_ref, g3_ref, g4_ref, g5_ref, dinv_ref,
                w_ref, b_ref, y_ref):
    w = w_ref[...]
    v0 = w[0] - w[2] + w[4]
    v1 = w[1] - w[3] + w[5]
    v2 = 2.0 * (w[2] - w[4])
    v3 = 2.0 * (w[3] - w[5])
    v4 = 2.0 * w[4]
    v5 = 2.0 * w[5]
    dot = lambda a, bm: jax.lax.dot(a, bm, preferred_element_type=_f32)
    accg = dot(g1_ref[0], v1)
    accg += dot(g2_ref[0], v2)
    accg += dot(g3_ref[0], v3)
    accg += dot(g4_ref[0], v4)
    accg += dot(g5_ref[0], v5)
    o = dot(x_ref[0], v0) + dinv_ref[...] * accg + b_ref[...]
    y_ref[0] = jnp.where(o > 0.0, o, jnp.exp(o) - 1.0)


_tc_final = pl.pallas_call(
    _final_body, grid=(NP_ // BN, B),
    in_specs=[_blk3(), _blk3(), _blk3(), _blk3(), _blk3(), _blk3(), _blk2(),
              pl.BlockSpec((6, F, F), lambda ni, bi: (0, 0, 0)),
              pl.BlockSpec((1, F), lambda ni, bi: (0, 0))],
    out_specs=_blk3(), out_shape=_out3)


def kernel(x, edge_index, down_row, down_col, down_val, W, b):
    row = edge_index[0]
    col = edge_index[1]

    pade = EP - E
    ar = jnp.arange(pade, dtype=jnp.int32)
    colp = jnp.concatenate([col, ar % N])
    rowp = jnp.concatenate([row, N + ar % (NP_ - N)])
    idx4 = jnp.stack(
        [colp.reshape(NS, NGRP, G, WND), rowp.reshape(NS, NGRP, G, WND)],
        axis=3)
    row3 = rowp.reshape(NS, NWIN, WND)
    x_pad = jnp.pad(x, ((0, 0), (0, NP_ - N), (0, 0)))

    deg2 = _sc_deg(row3)
    deg = deg2[0] + deg2[1]
    dinv = jnp.where(deg > 0.0, lax.rsqrt(deg), 0.0)
    dinv_b = jnp.broadcast_to(dinv[:, None], (NP_, F))
    c1_b = -(dinv_b * dinv_b)
    c2_b = 2.0 * c1_b
    zeros2 = jnp.zeros((NP_, F), _f32)

    v0 = _tc_mul2(x_pad, -dinv_b)
    g1 = _sc_prop(v0, idx4, zeros2)
    v1 = _tc_mul2(g1, c1_b)
    g2 = _sc_prop(v1, idx4, zeros2)
    v2 = _tc_mulsub(g2, c2_b, v0)
    g3 = _sc_prop(v2, idx4, zeros2)
    v3 = _tc_mulsub(g3, c2_b, v1)
    g4 = _sc_prop(v3, idx4, zeros2)
    v4 = _tc_mulsub(g4, c2_b, v2)
    g5 = _sc_prop(v4, idx4, zeros2)

    y = _tc_final(x_pad, g1, g2, g3, g4, g5, dinv_b, W, b.reshape(1, F))

    padp = PP - P
    arp = jnp.arange(padp, dtype=jnp.int32)
    pcol = jnp.concatenate([down_col, arp % N])
    prow = jnp.concatenate([down_row, M + arp % (MP_ - M)])
    pidx4 = jnp.stack(
        [pcol.reshape(NS, PWIN, PWND), prow.reshape(NS, PWIN, PWND)], axis=2)
    pval = jnp.concatenate([down_val, jnp.zeros((padp,), _f32)])
    pval4 = jnp.broadcast_to(pval[:, None], (PP, F)).reshape(NS, PWIN, PWND, F)

    pooled = _sc_pool(y, pidx4, pval4)
    return pooled[:, :M, :]
